# baseline (speedup 1.0000x reference)
import jax
import jax.numpy as jnp
from jax import lax
from jax.experimental import pallas as pl
from jax.experimental.pallas import tpu as pltpu
from jax.experimental.pallas import tpu_sc as plsc

NC = 2
NS = 16
NW = NC * NS
L = 16

ST = 128
NBUF = 2
PW = 128


def _build(B, D, V1):
    b_per_w = B // NW
    n_stages = b_per_w // ST
    mesh = plsc.VectorSubcoreMesh(
        core_axis_name="c", subcore_axis_name="s",
        num_cores=NC, num_subcores=NS)

    def body(idx_hbm, t1_hbm, t2_hbm, out_hbm,
             idx_v, q1_v, q2_v, s_v, o_v, tb, outbuf, sems):
        wid = lax.axis_index("s") * NC + lax.axis_index("c")
        base = wid * b_per_w

        pltpu.sync_copy(idx_hbm.at[pl.ds(base, b_per_w)], idx_v)

        for c in range(b_per_w // L):
            v = idx_v[pl.ds(c * L, L)]
            m = v < V1
            t = jnp.where(m, v, v - V1)
            pair = lax.shift_right_logical(t, 1)
            q1_v[c // 8, pl.ds((c % 8) * L, L)] = jnp.where(m, pair, 0)
            q2_v[c // 8, pl.ds((c % 8) * L, L)] = jnp.where(m, 0, pair)
            s_v[pl.ds(c * L, L)] = jnp.where(m, 0, 1)
            o_v[pl.ds(c * L, L)] = lax.shift_left(
                lax.bitwise_and(t, 1), 6)

        def fire(st, b):
            pltpu.async_copy(t1_hbm.at[q1_v.at[st]], tb.at[b, 0],
                             sems.at[b])
            pltpu.async_copy(t2_hbm.at[q2_v.at[st]], tb.at[b, 1],
                             sems.at[b])

        def drain(st, b):
            pltpu.make_async_copy(t1_hbm.at[q1_v.at[st]], tb.at[b, 0],
                                  sems.at[b]).wait()
            pltpu.make_async_copy(t2_hbm.at[q2_v.at[st]], tb.at[b, 1],
                                  sems.at[b]).wait()

        for b in range(NBUF):
            fire(b, b)

        for st in range(n_stages):
            b = st % NBUF
            drain(st, b)
            rbase = st * ST

            def extract_group(g, _, b=b, rbase=rbase):
                sv = s_v[pl.ds(rbase + g * L, L)]
                ov = o_v[pl.ds(rbase + g * L, L)]
                for i in range(L):
                    s = sv[i]
                    o = ov[i]
                    orow = rbase // 2 + g * (L // 2) + i // 2
                    for k in range(D // L):
                        outbuf[orow, pl.ds((i % 2) * D + k * L, L)] = (
                            tb[b, s, g * L + i, pl.ds(o + k * L, L)])
                return ()

            lax.fori_loop(0, ST // L, extract_group, (), unroll=False)
            nxt = st + NBUF
            if nxt < n_stages:
                fire(nxt, b)
        obase = pl.multiple_of(wid * (b_per_w // 2), 8)
        pltpu.sync_copy(outbuf, out_hbm.at[pl.ds(obase, b_per_w // 2)])

    return pl.kernel(
        body,
        out_type=jax.ShapeDtypeStruct((B // 2, PW), jnp.float32),
        mesh=mesh,
        scratch_types=[
            pltpu.VMEM((b_per_w,), jnp.int32),
            pltpu.VMEM((n_stages, ST), jnp.int32),
            pltpu.VMEM((n_stages, ST), jnp.int32),
            pltpu.VMEM((b_per_w,), jnp.int32),
            pltpu.VMEM((b_per_w,), jnp.int32),
            pltpu.VMEM((NBUF, 2, ST, PW), jnp.float32),
            pltpu.VMEM((b_per_w // 2, PW), jnp.float32),
            pltpu.SemaphoreType.DMA((NBUF,)),
        ],
    )


def kernel(indices, table1, table2):
    B = indices.shape[0]
    V1, D = table1.shape
    V2 = table2.shape[0]
    t1 = table1.reshape(V1 * D // PW, PW)
    t2 = table2.reshape(V2 * D // PW, PW)
    out = _build(B, D, V1)(indices.astype(jnp.int32), t1, t2)
    return out.reshape(B, D)


# speedup vs baseline: 2.7918x; 2.7918x over previous
import jax
import jax.numpy as jnp
from jax import lax
from jax.experimental import pallas as pl
from jax.experimental.pallas import tpu as pltpu
from jax.experimental.pallas import tpu_sc as plsc

NC = 2
NS = 16
NW = NC * NS
L = 16

ST = 128
NBUF = 2
PW = 128

CB = 2048


def _tc_merge(tt1, tt2, V):
    def body(a_ref, b_ref, m_ref):
        m_ref[...] = jnp.concatenate(
            [a_ref[...].T, b_ref[...].T], axis=1)

    grid = (V + CB - 1) // CB
    return pl.pallas_call(
        body,
        grid=(grid,),
        in_specs=[
            pl.BlockSpec((64, CB), lambda i: (0, i)),
            pl.BlockSpec((64, CB), lambda i: (0, i)),
        ],
        out_specs=pl.BlockSpec((CB, PW), lambda i: (i, 0)),
        out_shape=jax.ShapeDtypeStruct((V, PW), jnp.float32),
    )(tt1, tt2)


def _sc_gather(B, D, V1):
    b_per_w = B // NW
    n_stages = b_per_w // ST
    mesh = plsc.VectorSubcoreMesh(
        core_axis_name="c", subcore_axis_name="s",
        num_cores=NC, num_subcores=NS)

    def body(idx_hbm, m_hbm, out_hbm, idx_v, q_v, o_v, tb, outbuf, sems):
        wid = lax.axis_index("s") * NC + lax.axis_index("c")
        base = wid * b_per_w

        pltpu.sync_copy(idx_hbm.at[pl.ds(base, b_per_w)], idx_v)

        for c in range(b_per_w // L):
            v = idx_v[pl.ds(c * L, L)]
            m = v < V1
            q_v[c // 8, pl.ds((c % 8) * L, L)] = jnp.where(m, v, v - V1)
            o_v[pl.ds(c * L, L)] = jnp.where(m, 0, D)

        def fire(st, b):
            pltpu.async_copy(m_hbm.at[q_v.at[st]], tb.at[b], sems.at[b])

        def drain(st, b):
            pltpu.make_async_copy(m_hbm.at[q_v.at[st]], tb.at[b],
                                  sems.at[b]).wait()

        for b in range(NBUF):
            fire(b, b)

        for st in range(n_stages):
            b = st % NBUF
            drain(st, b)
            rbase = st * ST

            def extract_group(g, _, b=b, rbase=rbase):
                ov = o_v[pl.ds(rbase + g * L, L)]
                for i in range(L):
                    o = ov[i]
                    orow = rbase // 2 + g * (L // 2) + i // 2
                    for k in range(D // L):
                        outbuf[orow, pl.ds((i % 2) * D + k * L, L)] = (
                            tb[b, g * L + i, pl.ds(o + k * L, L)])
                return ()

            lax.fori_loop(0, ST // L, extract_group, (), unroll=False)
            nxt = st + NBUF
            if nxt < n_stages:
                fire(nxt, b)
        obase = pl.multiple_of(wid * (b_per_w // 2), 8)
        pltpu.sync_copy(outbuf, out_hbm.at[pl.ds(obase, b_per_w // 2)])

    return pl.kernel(
        body,
        out_type=jax.ShapeDtypeStruct((B // 2, PW), jnp.float32),
        mesh=mesh,
        scratch_types=[
            pltpu.VMEM((b_per_w,), jnp.int32),
            pltpu.VMEM((n_stages, ST), jnp.int32),
            pltpu.VMEM((b_per_w,), jnp.int32),
            pltpu.VMEM((NBUF, ST, PW), jnp.float32),
            pltpu.VMEM((b_per_w // 2, PW), jnp.float32),
            pltpu.SemaphoreType.DMA((NBUF,)),
        ],
    )


def kernel(indices, table1, table2):
    B = indices.shape[0]
    V1, D = table1.shape
    merged = _tc_merge(table1.T, table2.T, V1)
    out = _sc_gather(B, D, V1)(indices.astype(jnp.int32), merged)
    return out.reshape(B, D)


# speedup vs baseline: 3.8563x; 1.3813x over previous
import jax
import jax.numpy as jnp
from jax import lax
from jax.experimental import pallas as pl
from jax.experimental.pallas import tpu as pltpu
from jax.experimental.pallas import tpu_sc as plsc

NC = 2
NS = 16
NW = NC * NS
L = 16

ST = 128
NBUF = 2
PW = 128

CB = 8192


def _tc_merge(tt1, tt2, V):
    def body(a_ref, b_ref, m_ref):
        m_ref[...] = jnp.concatenate(
            [a_ref[...].T, b_ref[...].T], axis=1)

    grid = (V + CB - 1) // CB
    return pl.pallas_call(
        body,
        grid=(grid,),
        in_specs=[
            pl.BlockSpec((64, CB), lambda i: (0, i)),
            pl.BlockSpec((64, CB), lambda i: (0, i)),
        ],
        out_specs=pl.BlockSpec((CB, PW), lambda i: (i, 0)),
        out_shape=jax.ShapeDtypeStruct((V, PW), jnp.float32),
    )(tt1, tt2)


def _sc_gather(B, D, V1):
    b_per_w = B // NW
    n_stages = b_per_w // ST
    mesh = plsc.VectorSubcoreMesh(
        core_axis_name="c", subcore_axis_name="s",
        num_cores=NC, num_subcores=NS)

    def body(idx_hbm, m_hbm, out_hbm, idx_v, q_v, o_v, tb, outbuf, sems):
        wid = lax.axis_index("s") * NC + lax.axis_index("c")
        base = wid * b_per_w

        pltpu.sync_copy(idx_hbm.at[pl.ds(base, b_per_w)], idx_v)

        for c in range(b_per_w // L):
            v = idx_v[pl.ds(c * L, L)]
            m = v < V1
            q_v[c // 8, pl.ds((c % 8) * L, L)] = jnp.where(m, v, v - V1)
            o_v[pl.ds(c * L, L)] = jnp.where(m, 0, D)

        def fire(st, b):
            pltpu.async_copy(m_hbm.at[q_v.at[st]], tb.at[b], sems.at[b])

        def drain(st, b):
            pltpu.make_async_copy(m_hbm.at[q_v.at[st]], tb.at[b],
                                  sems.at[b]).wait()

        for b in range(NBUF):
            fire(b, b)

        for st in range(n_stages):
            b = st % NBUF
            drain(st, b)
            rbase = st * ST

            def extract_group(g, _, b=b, rbase=rbase):
                ov = o_v[pl.ds(rbase + g * L, L)]
                for i in range(L):
                    o = ov[i]
                    orow = rbase // 2 + g * (L // 2) + i // 2
                    for k in range(D // L):
                        outbuf[orow, pl.ds((i % 2) * D + k * L, L)] = (
                            tb[b, g * L + i, pl.ds(o + k * L, L)])
                return ()

            lax.fori_loop(0, ST // L, extract_group, (), unroll=False)
            nxt = st + NBUF
            if nxt < n_stages:
                fire(nxt, b)
        obase = pl.multiple_of(wid * (b_per_w // 2), 8)
        pltpu.sync_copy(outbuf, out_hbm.at[pl.ds(obase, b_per_w // 2)])

    return pl.kernel(
        body,
        out_type=jax.ShapeDtypeStruct((B // 2, PW), jnp.float32),
        mesh=mesh,
        scratch_types=[
            pltpu.VMEM((b_per_w,), jnp.int32),
            pltpu.VMEM((n_stages, ST), jnp.int32),
            pltpu.VMEM((b_per_w,), jnp.int32),
            pltpu.VMEM((NBUF, ST, PW), jnp.float32),
            pltpu.VMEM((b_per_w // 2, PW), jnp.float32),
            pltpu.SemaphoreType.DMA((NBUF,)),
        ],
    )


def kernel(indices, table1, table2):
    B = indices.shape[0]
    V1, D = table1.shape
    merged = _tc_merge(table1.T, table2.T, V1)
    out = _sc_gather(B, D, V1)(indices.astype(jnp.int32), merged)
    return out.reshape(B, D)


# speedup vs baseline: 4.0600x; 1.0528x over previous
import jax
import jax.numpy as jnp
from jax import lax
from jax.experimental import pallas as pl
from jax.experimental.pallas import tpu as pltpu
from jax.experimental.pallas import tpu_sc as plsc

NC = 2
NS = 16
NW = NC * NS
L = 16

ST = 128
NBUF = 2
PW = 128

CB = 16384


def _tc_merge(tt1, tt2, V):
    def body(a_ref, b_ref, m_ref):
        m_ref[...] = jnp.concatenate(
            [a_ref[...].T, b_ref[...].T], axis=1)

    grid = (V + CB - 1) // CB
    return pl.pallas_call(
        body,
        grid=(grid,),
        in_specs=[
            pl.BlockSpec((64, CB), lambda i: (0, i)),
            pl.BlockSpec((64, CB), lambda i: (0, i)),
        ],
        out_specs=pl.BlockSpec((CB, PW), lambda i: (i, 0)),
        out_shape=jax.ShapeDtypeStruct((V, PW), jnp.float32),
    )(tt1, tt2)


def _sc_gather(B, D, V1):
    b_per_w = B // NW
    n_stages = b_per_w // ST
    mesh = plsc.VectorSubcoreMesh(
        core_axis_name="c", subcore_axis_name="s",
        num_cores=NC, num_subcores=NS)

    def body(idx_hbm, m_hbm, out_hbm, idx_v, q_v, o_v, tb, outbuf, sems):
        wid = lax.axis_index("s") * NC + lax.axis_index("c")
        base = wid * b_per_w

        pltpu.sync_copy(idx_hbm.at[pl.ds(base, b_per_w)], idx_v)

        for c in range(b_per_w // L):
            v = idx_v[pl.ds(c * L, L)]
            m = v < V1
            q_v[c // 8, pl.ds((c % 8) * L, L)] = jnp.where(m, v, v - V1)
            o_v[pl.ds(c * L, L)] = jnp.where(m, 0, D)

        def fire(st, b):
            pltpu.async_copy(m_hbm.at[q_v.at[st]], tb.at[b], sems.at[b])

        def drain(st, b):
            pltpu.make_async_copy(m_hbm.at[q_v.at[st]], tb.at[b],
                                  sems.at[b]).wait()

        for b in range(NBUF):
            fire(b, b)

        for st in range(n_stages):
            b = st % NBUF
            drain(st, b)
            rbase = st * ST

            def extract_group(g, _, b=b, rbase=rbase):
                ov = o_v[pl.ds(rbase + g * L, L)]
                for i in range(L):
                    o = ov[i]
                    orow = rbase // 2 + g * (L // 2) + i // 2
                    for k in range(D // L):
                        outbuf[orow, pl.ds((i % 2) * D + k * L, L)] = (
                            tb[b, g * L + i, pl.ds(o + k * L, L)])
                return ()

            lax.fori_loop(0, ST // L, extract_group, (), unroll=False)
            nxt = st + NBUF
            if nxt < n_stages:
                fire(nxt, b)
        obase = pl.multiple_of(wid * (b_per_w // 2), 8)
        pltpu.sync_copy(outbuf, out_hbm.at[pl.ds(obase, b_per_w // 2)])

    return pl.kernel(
        body,
        out_type=jax.ShapeDtypeStruct((B // 2, PW), jnp.float32),
        mesh=mesh,
        scratch_types=[
            pltpu.VMEM((b_per_w,), jnp.int32),
            pltpu.VMEM((n_stages, ST), jnp.int32),
            pltpu.VMEM((b_per_w,), jnp.int32),
            pltpu.VMEM((NBUF, ST, PW), jnp.float32),
            pltpu.VMEM((b_per_w // 2, PW), jnp.float32),
            pltpu.SemaphoreType.DMA((NBUF,)),
        ],
    )


def kernel(indices, table1, table2):
    B = indices.shape[0]
    V1, D = table1.shape
    merged = _tc_merge(table1.T, table2.T, V1)
    out = _sc_gather(B, D, V1)(indices.astype(jnp.int32), merged)
    return out.reshape(B, D)
